# baseline (speedup 1.0000x reference)
import functools

import jax
import jax.numpy as jnp
from jax import lax
from jax.experimental import pallas as pl
from jax.experimental.pallas import tpu as pltpu

B = 4
R = 5000
C = 80
K_PRE = 1000
NMS_THR = 0.5
SCORE_THR = 0.05
IM_H = 1024.0
IM_W = 1024.0
CLS_OFF = 1026.0


def _decode_score_kernel(rois_ref, cls_ref, reg_ref, boxes_ref, s_ref,
                         cut_ref):
    scores = jax.nn.sigmoid(cls_ref[...])
    s = jnp.where(scores > SCORE_THR, scores, 0.0)
    s_ref[...] = s

    u = lax.bitcast_convert_type(s, jnp.int32)

    def bis(_, lohi):
        lo, hi = lohi
        mid = lo + (hi - lo) // 2
        cnt = jnp.sum((u > mid).astype(jnp.int32))
        big = cnt >= K_PRE
        return jnp.where(big, mid, lo), jnp.where(big, hi, mid)

    lo0 = jnp.int32(-1)
    hi0 = jnp.int32(0x3F800000)
    _, hi = lax.fori_loop(0, 32, bis, (lo0, hi0))
    cut_ref[...] = jnp.full((1, 1, 128), lax.bitcast_convert_type(hi, jnp.float32))

    rois = rois_ref[...]
    reg = reg_ref[...]
    w = rois[..., 2] - rois[..., 0]
    h = rois[..., 3] - rois[..., 1]
    cx = rois[..., 0] + 0.5 * w
    cy = rois[..., 1] + 0.5 * h
    dx = reg[..., 0]
    dy = reg[..., 1]
    dw = jnp.clip(reg[..., 2], -4.0, 4.0)
    dh = jnp.clip(reg[..., 3], -4.0, 4.0)
    pcx = cx + dx * w
    pcy = cy + dy * h
    pw = w * jnp.exp(dw)
    ph = h * jnp.exp(dh)
    x1 = jnp.clip(pcx - 0.5 * pw, 0.0, IM_W)
    y1 = jnp.clip(pcy - 0.5 * ph, 0.0, IM_H)
    x2 = jnp.clip(pcx + 0.5 * pw, 0.0, IM_W)
    y2 = jnp.clip(pcy + 0.5 * ph, 0.0, IM_H)
    boxes_ref[...] = jnp.stack([x1, y1, x2, y2], axis=-1)


def _nms_kernel(x1_ref, y1_ref, x2_ref, y2_ref, cls_ref, tbox_ref, s_ref,
                kept_ref, tboff_ref):
    off = cls_ref[...] * CLS_OFF
    x1 = x1_ref[...] + off
    y1 = y1_ref[...] + off
    x2 = x2_ref[...] + off
    y2 = y2_ref[...] + off
    area = (x2_ref[...] - x1_ref[...]) * (y2_ref[...] - y1_ref[...])

    coff = jnp.transpose(cls_ref[...])[:, :, None] * CLS_OFF
    tboff_ref[...] = tbox_ref[...] + coff

    lane = lax.broadcasted_iota(jnp.int32, (1, K_PRE), 1)

    def body(i, keep):
        tb = tboff_ref[i]
        x1i = tb[:, 0:1]
        y1i = tb[:, 1:2]
        x2i = tb[:, 2:3]
        y2i = tb[:, 3:4]
        area_i = (x2i - x1i) * (y2i - y1i)
        ix1 = jnp.maximum(x1i, x1)
        iy1 = jnp.maximum(y1i, y1)
        ix2 = jnp.minimum(x2i, x2)
        iy2 = jnp.minimum(y2i, y2)
        inter = jnp.maximum(ix2 - ix1, 0.0) * jnp.maximum(iy2 - iy1, 0.0)
        union = jnp.maximum(area_i + area - inter, 1e-6)
        onehot = (lane == i).astype(jnp.float32)
        keep_i = jnp.sum(keep * onehot, axis=1, keepdims=True)
        sup = ((inter > NMS_THR * union)
               & (lane > i)
               & (keep_i > 0.0))
        return jnp.where(sup, 0.0, keep)

    keep = lax.fori_loop(0, K_PRE, body, jnp.ones((B, K_PRE), jnp.float32))
    kept_ref[...] = s_ref[...] * keep


@jax.jit
def kernel(batch_rois, rcnn_cls_pred, rcnn_reg_pred):
    cls = rcnn_cls_pred[:, :, 0, 0].reshape(B, R, C)
    reg = rcnn_reg_pred[:, :, 0, 0].reshape(B, R, 4)

    boxes, s, cut = pl.pallas_call(
        _decode_score_kernel,
        grid=(B,),
        in_specs=[
            pl.BlockSpec((1, R, 4), lambda b: (b, 0, 0)),
            pl.BlockSpec((1, R, C), lambda b: (b, 0, 0)),
            pl.BlockSpec((1, R, 4), lambda b: (b, 0, 0)),
        ],
        out_specs=[
            pl.BlockSpec((1, R, 4), lambda b: (b, 0, 0)),
            pl.BlockSpec((1, R, C), lambda b: (b, 0, 0)),
            pl.BlockSpec((1, 1, 128), lambda b: (b, 0, 0)),
        ],
        out_shape=[
            jax.ShapeDtypeStruct((B, R, 4), jnp.float32),
            jax.ShapeDtypeStruct((B, R, C), jnp.float32),
            jax.ShapeDtypeStruct((B, 1, 128), jnp.float32),
        ],
    )(batch_rois, cls, reg)

    N = R * C
    M = 2 * K_PRE + 48
    s_flat = s.reshape(B, N)
    mask = s_flat >= cut[:, 0, :1]
    pos = jnp.where(mask, jnp.cumsum(mask.astype(jnp.int32), axis=1) - 1, M)
    bidx = jnp.broadcast_to(jnp.arange(B, dtype=jnp.int32)[:, None], (B, N))
    idx_full = jnp.broadcast_to(jnp.arange(N, dtype=jnp.int32)[None], (B, N))
    buf_i = jnp.full((B, M), N, jnp.int32).at[bidx, pos].set(
        idx_full, mode="drop")
    buf_s = jnp.where(
        buf_i < N,
        jnp.take_along_axis(s_flat, jnp.minimum(buf_i, N - 1), axis=1),
        0.0)
    top_s, tj = lax.top_k(buf_s, K_PRE)
    top_i = jnp.take_along_axis(buf_i, tj, axis=1)
    top_boxes = jnp.take_along_axis(boxes, (top_i // C)[:, :, None], axis=1)
    top_cls = (top_i % C + 1).astype(jnp.float32)

    tbox = jnp.transpose(top_boxes, (1, 0, 2))
    kept = pl.pallas_call(
        _nms_kernel,
        in_specs=[
            pl.BlockSpec((B, K_PRE), lambda: (0, 0)),
            pl.BlockSpec((B, K_PRE), lambda: (0, 0)),
            pl.BlockSpec((B, K_PRE), lambda: (0, 0)),
            pl.BlockSpec((B, K_PRE), lambda: (0, 0)),
            pl.BlockSpec((B, K_PRE), lambda: (0, 0)),
            pl.BlockSpec((K_PRE, B, 4), lambda: (0, 0, 0)),
            pl.BlockSpec((B, K_PRE), lambda: (0, 0)),
        ],
        out_specs=pl.BlockSpec((B, K_PRE), lambda: (0, 0)),
        out_shape=jax.ShapeDtypeStruct((B, K_PRE), jnp.float32),
        scratch_shapes=[pltpu.VMEM((K_PRE, B, 4), jnp.float32)],
    )(top_boxes[..., 0], top_boxes[..., 1], top_boxes[..., 2],
      top_boxes[..., 3], top_cls, tbox, top_s)

    return jnp.concatenate(
        [top_boxes, kept[..., None], top_cls[..., None]], axis=-1)


# speedup vs baseline: 3.9152x; 3.9152x over previous
import functools

import jax
import jax.numpy as jnp
from jax import lax
from jax.experimental import pallas as pl
from jax.experimental.pallas import tpu as pltpu

B = 4
R = 5000
C = 80
K_PRE = 1000
NMS_THR = 0.5
SCORE_THR = 0.05
IM_H = 1024.0
IM_W = 1024.0
CLS_OFF = 1026.0


def _decode_score_kernel(rois_ref, cls_ref, reg_ref, boxes_ref, s_ref):
    scores = jax.nn.sigmoid(cls_ref[...])
    s_ref[...] = jnp.where(scores > SCORE_THR, scores, 0.0)

    rois = rois_ref[...]
    reg = reg_ref[...]
    w = rois[..., 2] - rois[..., 0]
    h = rois[..., 3] - rois[..., 1]
    cx = rois[..., 0] + 0.5 * w
    cy = rois[..., 1] + 0.5 * h
    dx = reg[..., 0]
    dy = reg[..., 1]
    dw = jnp.clip(reg[..., 2], -4.0, 4.0)
    dh = jnp.clip(reg[..., 3], -4.0, 4.0)
    pcx = cx + dx * w
    pcy = cy + dy * h
    pw = w * jnp.exp(dw)
    ph = h * jnp.exp(dh)
    x1 = jnp.clip(pcx - 0.5 * pw, 0.0, IM_W)
    y1 = jnp.clip(pcy - 0.5 * ph, 0.0, IM_H)
    x2 = jnp.clip(pcx + 0.5 * pw, 0.0, IM_W)
    y2 = jnp.clip(pcy + 0.5 * ph, 0.0, IM_H)
    boxes_ref[...] = jnp.stack([x1, y1, x2, y2], axis=-1)


def _nms_kernel(x1_ref, y1_ref, x2_ref, y2_ref, cls_ref, tbox_ref, s_ref,
                kept_ref, tboff_ref, sup_ref):
    off = cls_ref[...] * CLS_OFF
    x1 = x1_ref[...] + off
    y1 = y1_ref[...] + off
    x2 = x2_ref[...] + off
    y2 = y2_ref[...] + off
    area = (x2_ref[...] - x1_ref[...]) * (y2_ref[...] - y1_ref[...])

    coff = jnp.transpose(cls_ref[...])[:, :, None] * CLS_OFF
    tboff_ref[...] = tbox_ref[...] + coff

    lane = lax.broadcasted_iota(jnp.int32, (1, K_PRE), 1)

    CH = 125

    def pre(c, _):
        i0 = c * CH
        tb = tboff_ref[pl.ds(i0, CH)]
        x1i = tb[:, :, 0:1]
        y1i = tb[:, :, 1:2]
        x2i = tb[:, :, 2:3]
        y2i = tb[:, :, 3:4]
        area_i = (x2i - x1i) * (y2i - y1i)
        ix1 = jnp.maximum(x1i, x1[None])
        iy1 = jnp.maximum(y1i, y1[None])
        ix2 = jnp.minimum(x2i, x2[None])
        iy2 = jnp.minimum(y2i, y2[None])
        inter = jnp.maximum(ix2 - ix1, 0.0) * jnp.maximum(iy2 - iy1, 0.0)
        union = jnp.maximum(area_i + area[None] - inter, 1e-6)
        ii = i0 + lax.broadcasted_iota(jnp.int32, (CH, 1, K_PRE), 0)
        jj = lax.broadcasted_iota(jnp.int32, (CH, 1, K_PRE), 2)
        sup = (inter > NMS_THR * union) & (jj > ii)
        sup_ref[pl.ds(i0, CH)] = sup.astype(jnp.float32)
        return 0

    lax.fori_loop(0, K_PRE // CH, pre, 0)

    def body(i, keep):
        row = sup_ref[i]
        onehot = (lane == i).astype(jnp.float32)
        keep_i = jnp.sum(keep * onehot, axis=1, keepdims=True)
        return jnp.where(row * keep_i > 0.0, 0.0, keep)

    keep = lax.fori_loop(0, K_PRE, body, jnp.ones((B, K_PRE), jnp.float32))
    kept_ref[...] = s_ref[...] * keep


@jax.jit
def kernel(batch_rois, rcnn_cls_pred, rcnn_reg_pred):
    cls = rcnn_cls_pred[:, :, 0, 0].reshape(B, R, C)
    reg = rcnn_reg_pred[:, :, 0, 0].reshape(B, R, 4)

    boxes, s = pl.pallas_call(
        _decode_score_kernel,
        grid=(B,),
        in_specs=[
            pl.BlockSpec((1, R, 4), lambda b: (b, 0, 0)),
            pl.BlockSpec((1, R, C), lambda b: (b, 0, 0)),
            pl.BlockSpec((1, R, 4), lambda b: (b, 0, 0)),
        ],
        out_specs=[
            pl.BlockSpec((1, R, 4), lambda b: (b, 0, 0)),
            pl.BlockSpec((1, R, C), lambda b: (b, 0, 0)),
        ],
        out_shape=[
            jax.ShapeDtypeStruct((B, R, 4), jnp.float32),
            jax.ShapeDtypeStruct((B, R, C), jnp.float32),
        ],
    )(batch_rois, cls, reg)

    s_flat = s.reshape(B, R * C)
    top_s, top_i = lax.top_k(s_flat, K_PRE)
    top_boxes = jnp.take_along_axis(boxes, (top_i // C)[:, :, None], axis=1)
    top_cls = (top_i % C + 1).astype(jnp.float32)

    tbox = jnp.transpose(top_boxes, (1, 0, 2))
    kept = pl.pallas_call(
        _nms_kernel,
        in_specs=[
            pl.BlockSpec((B, K_PRE), lambda: (0, 0)),
            pl.BlockSpec((B, K_PRE), lambda: (0, 0)),
            pl.BlockSpec((B, K_PRE), lambda: (0, 0)),
            pl.BlockSpec((B, K_PRE), lambda: (0, 0)),
            pl.BlockSpec((B, K_PRE), lambda: (0, 0)),
            pl.BlockSpec((K_PRE, B, 4), lambda: (0, 0, 0)),
            pl.BlockSpec((B, K_PRE), lambda: (0, 0)),
        ],
        out_specs=pl.BlockSpec((B, K_PRE), lambda: (0, 0)),
        out_shape=jax.ShapeDtypeStruct((B, K_PRE), jnp.float32),
        scratch_shapes=[pltpu.VMEM((K_PRE, B, 4), jnp.float32),
                        pltpu.VMEM((K_PRE, B, K_PRE), jnp.float32)],
    )(top_boxes[..., 0], top_boxes[..., 1], top_boxes[..., 2],
      top_boxes[..., 3], top_cls, tbox, top_s)

    return jnp.concatenate(
        [top_boxes, kept[..., None], top_cls[..., None]], axis=-1)
